# Initial kernel scaffold; baseline (speedup 1.0000x reference)
import jax
import jax.numpy as jnp
from jax.experimental import pallas as pl


def kernel(inventory, node_embeds, W, b):
    raise NotImplementedError("write your pallas kernel here")


# speedup vs baseline: 5.9094x; 5.9094x over previous
import functools

import jax
import jax.numpy as jnp
from jax import lax
from jax.experimental import pallas as pl
from jax.experimental.pallas import tpu as pltpu
from jax.experimental.pallas import tpu_sc as plsc

B = 16384
N = 50
D = 64
K = 64
OUT = 64

NC = 2
NS = 16
L = 16

HALF = OUT // 2
ROWS_PER_TEC = B // NS
CHUNK = 256
GROUPS = CHUNK // L
NCHUNK = ROWS_PER_TEC // CHUNK


def _tc_table_matmul(node_embeds, w_mat):
    def body(e_ref, w_ref, o_ref):
        o_ref[...] = jnp.dot(e_ref[...], w_ref[...],
                             preferred_element_type=jnp.float32)

    return pl.pallas_call(
        body,
        out_shape=jax.ShapeDtypeStruct((K, N * OUT), jnp.float32),
    )(node_embeds, w_mat)


_sc_mesh = plsc.VectorSubcoreMesh(
    core_axis_name="c", subcore_axis_name="s", num_cores=NC, num_subcores=NS)


@functools.partial(
    pl.kernel,
    out_type=jax.ShapeDtypeStruct((OUT, B), jnp.float32),
    mesh=_sc_mesh,
    compiler_params=pltpu.CompilerParams(needs_layout_passes=False),
    scratch_types=[
        pltpu.VMEM((N * K * HALF,), jnp.float32),
        pltpu.VMEM((CHUNK * N,), jnp.int32),
        pltpu.VMEM((HALF, CHUNK), jnp.float32),
    ],
)
def _sc_gather_sum(tbl_hbm, inv_hbm, out_hbm, tbl_v, inv_v, outb_v):
    cid = lax.axis_index("c")
    sid = lax.axis_index("s")
    rowbase = sid * ROWS_PER_TEC

    pltpu.sync_copy(tbl_hbm.at[pl.ds(cid * (N * K * HALF), N * K * HALF)],
                    tbl_v)

    lane = lax.iota(jnp.int32, L)
    lane_n = lane * N

    def chunk_body(ch, carry):
        row0 = rowbase + ch * CHUNK
        pltpu.sync_copy(inv_hbm.at[pl.ds(row0 * N, CHUNK * N)], inv_v)

        def group_body(g, carry2):
            def n_body(n, accs):
                idxv = plsc.load_gather(inv_v, [lane_n + (g * (L * N) + n)])
                base = idxv * HALF + n * (K * HALF)
                return [a + plsc.load_gather(tbl_v, [base + c])
                        for c, a in enumerate(accs)]

            accs = lax.fori_loop(
                0, N, n_body, [jnp.zeros((L,), jnp.float32)] * HALF)
            rowv = g * L + lane
            for c in range(HALF):
                plsc.store_scatter(
                    outb_v, [jnp.full((L,), c, jnp.int32), rowv], accs[c])
            return carry2

        lax.fori_loop(0, GROUPS, group_body, 0)
        pltpu.sync_copy(
            outb_v, out_hbm.at[pl.ds(cid * HALF, HALF), pl.ds(row0, CHUNK)])
        return carry

    lax.fori_loop(0, NCHUNK, chunk_body, 0)


def kernel(inventory, node_embeds, W, b):
    w_mat = W.reshape(OUT, N, D).transpose(2, 1, 0).reshape(D, N * OUT)
    p2 = _tc_table_matmul(node_embeds, w_mat)
    pt = p2.reshape(K, N, OUT).transpose(1, 0, 2).reshape(N * K, OUT)
    pt = pt.at[:K].add(b[None, :])
    tbl = jnp.concatenate(
        [pt[:, :HALF].reshape(-1), pt[:, HALF:].reshape(-1)])
    inv_flat = inventory.reshape(-1)
    out_t = _sc_gather_sum(tbl, inv_flat)
    return out_t.T


# speedup vs baseline: 8.4777x; 1.4346x over previous
import functools

import jax
import jax.numpy as jnp
from jax import lax
from jax.experimental import pallas as pl
from jax.experimental.pallas import tpu as pltpu
from jax.experimental.pallas import tpu_sc as plsc

B = 16384
N = 50
D = 64
K = 64
OUT = 64

NC = 2
NS = 16
L = 16

HALF = OUT // 2
ROWS_PER_TEC = B // NS
CHUNK = 256
GROUPS = CHUNK // L
NCHUNK = ROWS_PER_TEC // CHUNK


def _tc_table_matmul(node_embeds, w_mat):
    def body(e_ref, w_ref, o_ref):
        o_ref[...] = jnp.dot(e_ref[...], w_ref[...],
                             preferred_element_type=jnp.float32)

    return pl.pallas_call(
        body,
        out_shape=jax.ShapeDtypeStruct((K, N * OUT), jnp.float32),
    )(node_embeds, w_mat)


_sc_mesh = plsc.VectorSubcoreMesh(
    core_axis_name="c", subcore_axis_name="s", num_cores=NC, num_subcores=NS)


@functools.partial(
    pl.kernel,
    out_type=jax.ShapeDtypeStruct((OUT, B), jnp.float32),
    mesh=_sc_mesh,
    compiler_params=pltpu.CompilerParams(needs_layout_passes=False),
    scratch_types=[
        pltpu.VMEM((N * K * HALF,), jnp.float32),
        pltpu.VMEM((CHUNK * N,), jnp.int32),
        pltpu.VMEM((HALF, CHUNK), jnp.float32),
    ],
)
def _sc_gather_sum(tbl_hbm, inv_hbm, out_hbm, tbl_v, inv_v, outb_v):
    cid = lax.axis_index("c")
    sid = lax.axis_index("s")
    rowbase = sid * ROWS_PER_TEC

    pltpu.sync_copy(tbl_hbm.at[pl.ds(cid * (N * K * HALF), N * K * HALF)],
                    tbl_v)

    lane = lax.iota(jnp.int32, L)
    lane_n = lane * N

    def chunk_body(ch, carry):
        row0 = rowbase + ch * CHUNK
        pltpu.sync_copy(inv_hbm.at[pl.ds(row0 * N, CHUNK * N)], inv_v)

        def group_body(g, carry2):
            def n_body(n, accs):
                idxv = plsc.load_gather(inv_v, [lane_n + (g * (L * N) + n)])
                base = idxv * HALF + n * (K * HALF)
                return [a + plsc.load_gather(tbl_v, [base | (lane ^ c)])
                        for c, a in enumerate(accs)]

            accs = lax.fori_loop(
                0, N, n_body, [jnp.zeros((L,), jnp.float32)] * HALF)
            rowv = g * L + lane
            for c in range(HALF):
                plsc.store_scatter(
                    outb_v, [lane ^ c, rowv], accs[c])
            return carry2

        lax.fori_loop(0, GROUPS, group_body, 0)
        pltpu.sync_copy(
            outb_v, out_hbm.at[pl.ds(cid * HALF, HALF), pl.ds(row0, CHUNK)])
        return carry

    lax.fori_loop(0, NCHUNK, chunk_body, 0)


def kernel(inventory, node_embeds, W, b):
    w_mat = W.reshape(OUT, N, D).transpose(2, 1, 0).reshape(D, N * OUT)
    p2 = _tc_table_matmul(node_embeds, w_mat)
    pt = p2.reshape(K, N, OUT).transpose(1, 0, 2).reshape(N * K, OUT)
    pt = pt.at[:K].add(b[None, :])
    tbl = jnp.concatenate(
        [pt[:, :HALF].reshape(-1), pt[:, HALF:].reshape(-1)])
    inv_flat = inventory.reshape(-1)
    out_t = _sc_gather_sum(tbl, inv_flat)
    return out_t.T


# speedup vs baseline: 30.7593x; 3.6283x over previous
import functools

import jax
import jax.numpy as jnp
from jax import lax
from jax.experimental import pallas as pl
from jax.experimental.pallas import tpu as pltpu
from jax.experimental.pallas import tpu_sc as plsc

B = 16384
N = 50
D = 64
K = 64
OUT = 64

NC = 2
NS = 16
L = 16

HALF = OUT // 2
ROWS_PER_TEC = B // NS
CHUNK = 256
GROUPS = CHUNK // L
NCHUNK = ROWS_PER_TEC // CHUNK


def _tc_table_matmul(node_embeds, w_mat):
    def body(e_ref, w_ref, o_ref):
        o_ref[...] = jnp.dot(e_ref[...], w_ref[...],
                             preferred_element_type=jnp.float32)

    return pl.pallas_call(
        body,
        out_shape=jax.ShapeDtypeStruct((K, N * OUT), jnp.float32),
    )(node_embeds, w_mat)


_sc_mesh = plsc.VectorSubcoreMesh(
    core_axis_name="c", subcore_axis_name="s", num_cores=NC, num_subcores=NS)


@functools.partial(
    pl.kernel,
    out_type=jax.ShapeDtypeStruct((OUT, B), jnp.float32),
    mesh=_sc_mesh,
    compiler_params=pltpu.CompilerParams(needs_layout_passes=False),
    scratch_types=[
        pltpu.VMEM((N * K * HALF,), jnp.float32),
        pltpu.VMEM((CHUNK * N,), jnp.int32),
        pltpu.VMEM((HALF, CHUNK), jnp.float32),
    ],
)
def _sc_gather_sum(tbl_hbm, inv_hbm, out_hbm, tbl_v, inv_v, outb_v):
    cid = lax.axis_index("c")
    sid = lax.axis_index("s")
    rowbase = sid * ROWS_PER_TEC

    pltpu.sync_copy(tbl_hbm.at[pl.ds(cid * (N * K * HALF), N * K * HALF)],
                    tbl_v)

    lane = lax.iota(jnp.int32, L)
    lane_n = lane * N

    def chunk_body(ch, carry):
        row0 = rowbase + ch * CHUNK
        pltpu.sync_copy(inv_hbm.at[pl.ds(row0 * N, CHUNK * N)], inv_v)

        def group_body(g, carry2):
            rowv = g * L + lane
            for p in range(2):
                def n_body(n, accs):
                    idxv = plsc.load_gather(
                        inv_v, [lane_n + (g * (L * N) + n)])
                    base = idxv * HALF + (n * (K * HALF) + p * L)
                    return [a + plsc.load_gather(tbl_v, [base | (lane ^ c)])
                            for c, a in enumerate(accs)]

                accs = lax.fori_loop(
                    0, N, n_body, [jnp.zeros((L,), jnp.float32)] * L)
                for c in range(L):
                    plsc.store_scatter(
                        outb_v, [p * L + (lane ^ c), rowv], accs[c])
            return carry2

        lax.fori_loop(0, GROUPS, group_body, 0)
        pltpu.sync_copy(
            outb_v, out_hbm.at[pl.ds(cid * HALF, HALF), pl.ds(row0, CHUNK)])
        return carry

    lax.fori_loop(0, NCHUNK, chunk_body, 0)


def kernel(inventory, node_embeds, W, b):
    w_mat = W.reshape(OUT, N, D).transpose(2, 1, 0).reshape(D, N * OUT)
    p2 = _tc_table_matmul(node_embeds, w_mat)
    pt = p2.reshape(K, N, OUT).transpose(1, 0, 2).reshape(N * K, OUT)
    pt = pt.at[:K].add(b[None, :])
    tbl = jnp.concatenate(
        [pt[:, :HALF].reshape(-1), pt[:, HALF:].reshape(-1)])
    inv_flat = inventory.reshape(-1)
    out_t = _sc_gather_sum(tbl, inv_flat)
    return out_t.T


# speedup vs baseline: 36.0102x; 1.1707x over previous
import functools

import jax
import jax.numpy as jnp
from jax import lax
from jax.experimental import pallas as pl
from jax.experimental.pallas import tpu as pltpu
from jax.experimental.pallas import tpu_sc as plsc

B = 16384
N = 50
D = 64
K = 64
OUT = 64

NC = 2
NS = 16
L = 16

HALF = OUT // 2
ROWS_PER_TEC = B // NS
CHUNK = 128
GROUPS = CHUNK // L
NCHUNK = ROWS_PER_TEC // CHUNK


def _tc_table_halves(node_embeds, w, b2):
    def body(e_ref, w_ref, b_ref, lo_ref, hi_ref):
        e = e_ref[...]
        bias = b_ref[...]
        for n in range(N):
            w_blk = w_ref[:, n * D:(n + 1) * D]
            p = lax.dot_general(e, w_blk, (((1,), (1,)), ((), ())),
                                preferred_element_type=jnp.float32)
            if n == 0:
                p = p + bias
            lo_ref[pl.ds(n * K, K), :] = p[:, :HALF]
            hi_ref[pl.ds(n * K, K), :] = p[:, HALF:]

    return pl.pallas_call(
        body,
        out_shape=(
            jax.ShapeDtypeStruct((N * K, HALF), jnp.float32),
            jax.ShapeDtypeStruct((N * K, HALF), jnp.float32),
        ),
    )(node_embeds, w, b2)


_sc_mesh = plsc.VectorSubcoreMesh(
    core_axis_name="c", subcore_axis_name="s", num_cores=NC, num_subcores=NS)


@functools.partial(
    pl.kernel,
    out_type=jax.ShapeDtypeStruct((OUT, B), jnp.float32),
    mesh=_sc_mesh,
    compiler_params=pltpu.CompilerParams(needs_layout_passes=False),
    scratch_types=[
        pltpu.VMEM((N * K * HALF,), jnp.float32),
        pltpu.VMEM((N, CHUNK), jnp.int32),
        pltpu.VMEM((N, CHUNK), jnp.int32),
        pltpu.VMEM((HALF, CHUNK), jnp.float32),
        pltpu.VMEM((HALF, CHUNK), jnp.float32),
        pltpu.SemaphoreType.DMA,
        pltpu.SemaphoreType.DMA,
        pltpu.SemaphoreType.DMA,
        pltpu.SemaphoreType.DMA,
        pltpu.SemaphoreType.DMA,
    ],
)
def _sc_gather_sum(lo_hbm, hi_hbm, invt_hbm, out_hbm,
                   tbl_v, inv_v0, inv_v1, outb_v0, outb_v1,
                   tbl_sem, i_sem0, i_sem1, o_sem0, o_sem1):
    cid = lax.axis_index("c")
    sid = lax.axis_index("s")
    rowbase = sid * ROWS_PER_TEC
    inv_bufs = (inv_v0, inv_v1)
    outb_bufs = (outb_v0, outb_v1)
    i_sems = (i_sem0, i_sem1)
    o_sems = (o_sem0, o_sem1)

    @pl.when(cid == 0)
    def _():
        pltpu.async_copy(lo_hbm, tbl_v, tbl_sem)

    @pl.when(cid == 1)
    def _():
        pltpu.async_copy(hi_hbm, tbl_v, tbl_sem)

    def inv_copy(ch):
        row0 = rowbase + ch * CHUNK
        return pltpu.async_copy(
            invt_hbm.at[:, pl.ds(row0, CHUNK)], inv_bufs[ch % 2],
            i_sems[ch % 2])

    inv_cps = [inv_copy(0)]
    pltpu.make_async_copy(lo_hbm, tbl_v, tbl_sem).wait()

    lane = lax.iota(jnp.int32, L)
    out_cps = [None, None]

    for ch in range(NCHUNK):
        buf = ch % 2
        inv_cps[ch].wait()
        if ch + 1 < NCHUNK:
            inv_cps.append(inv_copy(ch + 1))
        if out_cps[buf] is not None:
            out_cps[buf].wait()

        def group_body(g, carry, buf=buf):
            rowv = g * L + lane
            for p in range(2):
                def n_body(n, accs, p=p):
                    idxv = inv_bufs[buf][n, pl.ds(g * L, L)]
                    base = idxv * HALF + (n * (K * HALF) + p * L)
                    return [a + plsc.load_gather(tbl_v, [base | (lane ^ c)])
                            for c, a in enumerate(accs)]

                accs = lax.fori_loop(
                    0, N, n_body, [jnp.zeros((L,), jnp.float32)] * L,
                    unroll=2)
                for c in range(L):
                    plsc.store_scatter(
                        outb_bufs[buf], [p * L + (lane ^ c), rowv], accs[c])
            return carry

        lax.fori_loop(0, GROUPS, group_body, 0)
        row0 = rowbase + ch * CHUNK
        out_cps[buf] = pltpu.async_copy(
            outb_bufs[buf],
            out_hbm.at[pl.ds(cid * HALF, HALF), pl.ds(row0, CHUNK)],
            o_sems[buf])

    out_cps[0].wait()
    out_cps[1].wait()


def kernel(inventory, node_embeds, W, b):
    lo, hi = _tc_table_halves(node_embeds, W, b.reshape(1, OUT))
    out_t = _sc_gather_sum(lo.reshape(-1), hi.reshape(-1),
                           inventory.T)
    return out_t.T


# speedup vs baseline: 45.5444x; 1.2648x over previous
import functools

import jax
import jax.numpy as jnp
from jax import lax
from jax.experimental import pallas as pl
from jax.experimental.pallas import tpu as pltpu
from jax.experimental.pallas import tpu_sc as plsc

B = 16384
N = 50
D = 64
K = 64
OUT = 64

NC = 2
NS = 16
L = 16

WORDS = OUT // 2
ROWS_PER_TEC = B // (NC * NS)
CHUNK = 128
GROUPS = CHUNK // L
NCHUNK = ROWS_PER_TEC // CHUNK
PASSES = 4
WPP = WORDS // PASSES


def _tc_table(node_embeds, w, b2):
    def body(e_ref, w_ref, b_ref, o_ref):
        e = e_ref[...]
        bias = b_ref[...]
        for n in range(N):
            w_blk = w_ref[:, n * D:(n + 1) * D]
            p = lax.dot_general(e, w_blk, (((1,), (1,)), ((), ())),
                                preferred_element_type=jnp.float32)
            if n == 0:
                p = p + bias
            o_ref[pl.ds(n * K, K), :] = p.astype(jnp.bfloat16)

    return pl.pallas_call(
        body,
        out_shape=jax.ShapeDtypeStruct((N * K, OUT), jnp.bfloat16),
    )(node_embeds, w, b2)


_sc_mesh = plsc.VectorSubcoreMesh(
    core_axis_name="c", subcore_axis_name="s", num_cores=NC, num_subcores=NS)


@functools.partial(
    pl.kernel,
    out_type=jax.ShapeDtypeStruct((OUT, B), jnp.float32),
    mesh=_sc_mesh,
    compiler_params=pltpu.CompilerParams(needs_layout_passes=False),
    scratch_types=[
        pltpu.VMEM((N * K * WORDS,), jnp.int32),
        pltpu.VMEM((N, CHUNK), jnp.int32),
        pltpu.VMEM((N, CHUNK), jnp.int32),
        pltpu.VMEM((OUT, CHUNK), jnp.float32),
        pltpu.SemaphoreType.DMA,
        pltpu.SemaphoreType.DMA,
        pltpu.SemaphoreType.DMA,
        pltpu.SemaphoreType.DMA,
    ],
)
def _sc_gather_sum(tbl_hbm, invt_hbm, out_hbm,
                   tbl_v, inv_v0, inv_v1, outb_v,
                   tbl_sem, i_sem0, i_sem1, o_sem):
    cid = lax.axis_index("c")
    sid = lax.axis_index("s")
    wid = sid * NC + cid
    rowbase = wid * ROWS_PER_TEC
    inv_bufs = (inv_v0, inv_v1)
    i_sems = (i_sem0, i_sem1)

    tbl_cp = pltpu.async_copy(tbl_hbm, tbl_v, tbl_sem)

    def inv_copy(ch):
        row0 = rowbase + ch * CHUNK
        return pltpu.async_copy(
            invt_hbm.at[:, pl.ds(row0, CHUNK)], inv_bufs[ch % 2],
            i_sems[ch % 2])

    inv_cps = [inv_copy(0)]
    tbl_cp.wait()

    lane = lax.iota(jnp.int32, L)
    hi_mask = jnp.full((L,), -65536, jnp.int32)
    out_cp = None

    for ch in range(NCHUNK):
        buf = ch % 2
        inv_cps[ch].wait()
        if ch + 1 < NCHUNK:
            inv_cps.append(inv_copy(ch + 1))
        if out_cp is not None:
            out_cp.wait()

        def group_body(g, carry, buf=buf):
            rowv = g * L + lane
            for p in range(PASSES):
                def n_body(n, accs, p=p):
                    idxv = inv_bufs[buf][n, pl.ds(g * L, L)]
                    bl = ((idxv * WORDS) ^ lane) + n * (K * WORDS)
                    accs = list(accs)
                    for w8 in range(WPP):
                        word = plsc.load_gather(tbl_v, [bl ^ (p * WPP + w8)])
                        accs[2 * w8] = accs[2 * w8] + plsc.bitcast(
                            word << 16, jnp.float32)
                        accs[2 * w8 + 1] = accs[2 * w8 + 1] + plsc.bitcast(
                            word & hi_mask, jnp.float32)
                    return accs

                accs = lax.fori_loop(
                    0, N, n_body, [jnp.zeros((L,), jnp.float32)] * L,
                    unroll=2)
                for w8 in range(WPP):
                    col = ((p * WPP + w8) ^ lane) * 2
                    plsc.store_scatter(outb_v, [col, rowv], accs[2 * w8])
                    plsc.store_scatter(outb_v, [col + 1, rowv],
                                       accs[2 * w8 + 1])
            return carry

        lax.fori_loop(0, GROUPS, group_body, 0)
        row0 = rowbase + ch * CHUNK
        out_cp = pltpu.async_copy(
            outb_v, out_hbm.at[:, pl.ds(row0, CHUNK)], o_sem)

    out_cp.wait()


def kernel(inventory, node_embeds, W, b):
    tbl16 = _tc_table(node_embeds, W, b.reshape(1, OUT))
    tbl = lax.bitcast_convert_type(
        tbl16.reshape(N * K, WORDS, 2), jnp.int32).reshape(-1)
    out_t = _sc_gather_sum(tbl, inventory.T)
    return out_t.T
